# baseline (speedup 1.0000x reference)
import functools

import jax
import jax.numpy as jnp
from jax import lax
from jax.experimental import pallas as pl
from jax.experimental.pallas import tpu as pltpu
from jax.experimental.pallas import tpu_sc as plsc

_B = 16384
_SUB = 256
_NCHUNK = _B // _SUB
_GRP = _SUB // 16
_L = 50
_DIXP = 15
_DFAC = 20
_WIXP = 9
_WFAC = 11
_BLK = 2048

_CAT_OFFS = [0, 8, 16, 32, 40, 48, 52, 56, 60]

_SC_PARAMS = pltpu.CompilerParams(
    needs_layout_passes=False, use_tc_tiling_on_sc=False)

_HI = jnp.uint32(0xFFFF0000)


def _unpack2(word_f32):
    w = plsc.bitcast(word_f32, jnp.uint32)
    even = plsc.bitcast(w << 16, jnp.float32)
    odd = plsc.bitcast(w & _HI, jnp.float32)
    return even, odd


def _sc_body(emb_u, emb_i, uidx2, iidx2, t1x, t2x, t1f, t2f,
             ix1, ix2, if1, if2,
             u_out, v_out, s1x_out, s2x_out, s1f_out, s2f_out,
             tix_v, tfac_v, idx_v, rows_v, it_v, soutx_v, soutf_v, sem):
    c = lax.axis_index("c")
    s = lax.axis_index("s")
    wid = s * 2 + c

    for (emb, midx2, tixp, ixraw, mout, sout) in (
            (emb_u, uidx2, t1x, ix1, u_out, s1x_out),
            (emb_i, iidx2, t2x, ix2, v_out, s2x_out)):
        pltpu.sync_copy(tixp, tix_v)
        pltpu.sync_copy(midx2.at[pl.ds(wid * 4, 4)], idx_v)
        cps = [pltpu.async_copy(emb.at[idx_v.at[mc]],
                                rows_v.at[pl.ds(mc * 128, 128)], sem)
               for mc in range(4)]

        for sub in range(2):
            chunk = wid * 2 + sub
            pltpu.sync_copy(ixraw.at[chunk], it_v)

            def group(g, carry):
                o = g * 16

                def kstep(k2, accs):
                    accs = list(accs)
                    for dk in range(2):
                        k = k2 * 2 + dk
                        rbase = it_v[k, pl.ds(o, 16)] * _WIXP
                        for jp in range(8):
                            ev, od = _unpack2(
                                plsc.load_gather(tix_v, [rbase + jp]))
                            accs[2 * jp] = accs[2 * jp] + ev
                            if 2 * jp + 1 < _DIXP:
                                accs[2 * jp + 1] = accs[2 * jp + 1] + od
                    return tuple(accs)

                accs = lax.fori_loop(
                    0, _L // 2, kstep,
                    tuple(jnp.zeros((16,), jnp.float32)
                          for _ in range(_DIXP)))
                for j in range(_DIXP):
                    soutx_v[j, pl.ds(o, 16)] = accs[j]
                soutx_v[_DIXP, pl.ds(o, 16)] = jnp.zeros((16,), jnp.float32)
                return carry

            lax.fori_loop(0, _GRP, group, 0)
            pltpu.sync_copy(soutx_v, sout.at[chunk])

        for cp in cps:
            cp.wait()
        pltpu.sync_copy(rows_v, mout.at[pl.ds(wid * 512, 512)])

    def fac_side(tfac, ifraw, sout):
        pltpu.sync_copy(tfac, tfac_v)

        def do_sub(sub, carry0):
            chunk = s * 4 + sub
            pltpu.sync_copy(ifraw.at[chunk], it_v)

            def group(g, carry):
                o = g * 16

                def kstep(k2, accs):
                    accs = list(accs)
                    for dk in range(2):
                        k = k2 * 2 + dk
                        rbase = it_v[k, pl.ds(o, 16)] * _WFAC
                        for jp in range(10):
                            ev, od = _unpack2(
                                plsc.load_gather(tfac_v, [rbase + jp]))
                            accs[2 * jp] = accs[2 * jp] + ev
                            accs[2 * jp + 1] = accs[2 * jp + 1] + od
                    return tuple(accs)

                accs = lax.fori_loop(
                    0, _L // 2, kstep,
                    tuple(jnp.zeros((16,), jnp.float32)
                          for _ in range(_DFAC)))
                for j in range(_DFAC):
                    soutf_v[j, pl.ds(o, 16)] = accs[j]
                for j in range(_DFAC, 24):
                    soutf_v[j, pl.ds(o, 16)] = jnp.zeros((16,), jnp.float32)
                return carry

            lax.fori_loop(0, _GRP, group, 0)
            pltpu.sync_copy(soutf_v, sout.at[chunk])
            return carry0

        lax.fori_loop(0, 4, do_sub, 0)

    @pl.when(c == 0)
    def _():
        fac_side(t1f, if1, s1f_out)

    @pl.when(c == 1)
    def _():
        fac_side(t2f, if2, s2f_out)


def _sc_main(emb_u, emb_i, uidx2, iidx2, t1x, t2x, t1f, t2f,
             ix1, ix2, if1, if2):
    mesh = plsc.VectorSubcoreMesh(core_axis_name="c", subcore_axis_name="s")
    f = functools.partial(
        pl.kernel, mesh=mesh, compiler_params=_SC_PARAMS,
        out_type=(
            jax.ShapeDtypeStruct((_B, 64), jnp.float32),
            jax.ShapeDtypeStruct((_B, 64), jnp.float32),
            jax.ShapeDtypeStruct((_NCHUNK, _DIXP + 1, _SUB), jnp.float32),
            jax.ShapeDtypeStruct((_NCHUNK, _DIXP + 1, _SUB), jnp.float32),
            jax.ShapeDtypeStruct((_NCHUNK, 24, _SUB), jnp.float32),
            jax.ShapeDtypeStruct((_NCHUNK, 24, _SUB), jnp.float32),
        ),
        scratch_types=(
            pltpu.VMEM((1000 * _WIXP,), jnp.float32),
            pltpu.VMEM((5000 * _WFAC,), jnp.float32),
            pltpu.VMEM((4, 128), jnp.int32),
            pltpu.VMEM((512, 64), jnp.float32),
            pltpu.VMEM((_L, _SUB), jnp.int32),
            pltpu.VMEM((_DIXP + 1, _SUB), jnp.float32),
            pltpu.VMEM((24, _SUB), jnp.float32),
            pltpu.SemaphoreType.DMA,
        ),
    )(_sc_body)
    return f(emb_u, emb_i, uidx2, iidx2, t1x, t2x, t1f, t2f,
             ix1, ix2, if1, if2)


def _tc_body(u_ref, v_ref, s1x_ref, s2x_ref, s1f_ref, s2f_ref,
             c1_ref, c2_ref, t1b_ref, t2b_ref,
             wmain_ref, wcat_ref, wixp_ref, wfac_ref, b_ref, out_ref):
    u = u_ref[...]
    v = v_ref[...]
    mterm = jnp.sum(u * v * wmain_ref[0, :][None, :], axis=1)

    lane = lax.broadcasted_iota(jnp.int32, (_BLK, 64), 1)
    oh1 = jnp.zeros((_BLK, 64), jnp.float32)
    oh2 = jnp.zeros((_BLK, 64), jnp.float32)
    for ci in range(9):
        off = _CAT_OFFS[ci]
        oh1 = oh1 + (lane == (c1_ref[:, ci][:, None] + off)).astype(jnp.float32)
        oh2 = oh2 + (lane == (c2_ref[:, ci][:, None] + off)).astype(jnp.float32)
    ucat = jnp.dot(oh1, t1b_ref[...], preferred_element_type=jnp.float32)
    vcat = jnp.dot(oh2, t2b_ref[...], preferred_element_type=jnp.float32)
    cterm = jnp.sum(ucat * vcat * wcat_ref[0, :][None, :], axis=1)

    xterm = jnp.sum(s1x_ref[...] * s2x_ref[...] * wixp_ref[...],
                    axis=0) * (1.0 / 2500.0)
    fterm = jnp.sum(s1f_ref[...] * s2f_ref[...] * wfac_ref[...],
                    axis=0) * (1.0 / 2500.0)

    out_ref[...] = mterm + cterm + xterm + fterm + b_ref[0, 0]


def _tc(u_main, v_main, s1x, s2x, s1f, s2f, c1, c2, t1b, t2b,
        wmain, wcat, wixp, wfac, b):
    grid = (_B // _BLK,)
    return pl.pallas_call(
        _tc_body,
        grid=grid,
        in_specs=[
            pl.BlockSpec((_BLK, 64), lambda i: (i, 0)),
            pl.BlockSpec((_BLK, 64), lambda i: (i, 0)),
            pl.BlockSpec((_DIXP + 1, _BLK), lambda i: (0, i)),
            pl.BlockSpec((_DIXP + 1, _BLK), lambda i: (0, i)),
            pl.BlockSpec((24, _BLK), lambda i: (0, i)),
            pl.BlockSpec((24, _BLK), lambda i: (0, i)),
            pl.BlockSpec((_BLK, 16), lambda i: (i, 0)),
            pl.BlockSpec((_BLK, 16), lambda i: (i, 0)),
            pl.BlockSpec((64, 64), lambda i: (0, 0)),
            pl.BlockSpec((64, 64), lambda i: (0, 0)),
            pl.BlockSpec((1, 64), lambda i: (0, 0)),
            pl.BlockSpec((1, 64), lambda i: (0, 0)),
            pl.BlockSpec((_DIXP + 1, 1), lambda i: (0, 0)),
            pl.BlockSpec((24, 1), lambda i: (0, 0)),
            pl.BlockSpec((1, 1), lambda i: (0, 0)),
        ],
        out_specs=pl.BlockSpec((_BLK,), lambda i: (i,)),
        out_shape=jax.ShapeDtypeStruct((_B,), jnp.float32),
    )(u_main, v_main, s1x, s2x, s1f, s2f, c1, c2, t1b, t2b,
      wmain, wcat, wixp, wfac, b)


def _untranspose(s3, rows):
    return jnp.transpose(s3, (1, 0, 2)).reshape(rows, _B)


def _chunked(idx2d):
    t = jnp.transpose(idx2d.astype(jnp.int32), (1, 0))
    t = t.reshape(_L, _NCHUNK, _SUB)
    return jnp.transpose(t, (1, 0, 2))


def _pack_bf16(tab, words):
    n, d = tab.shape
    if d % 2:
        tab = jnp.pad(tab, ((0, 0), (0, 1)))
        d += 1
    ev = jax.lax.bitcast_convert_type(
        tab[:, 0::2].astype(jnp.bfloat16), jnp.uint16).astype(jnp.uint32)
    od = jax.lax.bitcast_convert_type(
        tab[:, 1::2].astype(jnp.bfloat16), jnp.uint16).astype(jnp.uint32)
    w = jax.lax.bitcast_convert_type(ev | (od << 16), jnp.float32)
    w = jnp.pad(w, ((0, 0), (0, words - d // 2)))
    return w.reshape(-1)


def kernel(user_indices, item_indices, ASnode1_info_type, ASnode1_AS_tier, ASnode1_info_traffic, ASnode1_info_ratio, ASnode1_info_scope, ASnode1_policy_general, ASnode1_policy_locations, ASnode1_policy_ratio, ASnode1_policy_contracts, ASnode1_appearIXP, ASnode1_appearFac, ASnode2_info_type, ASnode2_AS_tier, ASnode2_info_traffic, ASnode2_info_ratio, ASnode2_info_scope, ASnode2_policy_general, ASnode2_policy_locations, ASnode2_policy_ratio, ASnode2_policy_contracts, ASnode2_appearIXP, ASnode2_appearFac, emb_user, emb_item, t1_info_type, t1_AS_tier, t1_info_traffic, t1_info_ratio, t1_info_scope, t1_policy_general, t1_policy_locations, t1_policy_ratio, t1_policy_contracts, t1_appearIXP, t1_appearFac, t2_info_type, t2_AS_tier, t2_info_traffic, t2_info_ratio, t2_info_scope, t2_policy_general, t2_policy_locations, t2_policy_ratio, t2_policy_contracts, t2_appearIXP, t2_appearFac, W, b):
    uidx2 = user_indices.astype(jnp.int32).reshape(_B // 128, 128)
    iidx2 = item_indices.astype(jnp.int32).reshape(_B // 128, 128)
    ix1 = _chunked(ASnode1_appearIXP)
    ix2 = _chunked(ASnode2_appearIXP)
    if1 = _chunked(ASnode1_appearFac)
    if2 = _chunked(ASnode2_appearFac)
    t1x = _pack_bf16(t1_appearIXP, _WIXP)
    t2x = _pack_bf16(t2_appearIXP, _WIXP)
    t1f = _pack_bf16(t1_appearFac, _WFAC)
    t2f = _pack_bf16(t2_appearFac, _WFAC)

    u_main, v_main, s1x3, s2x3, s1f3, s2f3 = _sc_main(
        emb_user, emb_item, uidx2, iidx2, t1x, t2x, t1f, t2f,
        ix1, ix2, if1, if2)
    s1x = _untranspose(s1x3, _DIXP + 1)
    s2x = _untranspose(s2x3, _DIXP + 1)
    s1f = _untranspose(s1f3, 24)
    s2f = _untranspose(s2f3, 24)

    cats1 = [ASnode1_info_type, ASnode1_AS_tier, ASnode1_info_traffic, ASnode1_info_ratio, ASnode1_info_scope, ASnode1_policy_general, ASnode1_policy_locations, ASnode1_policy_ratio, ASnode1_policy_contracts]
    cats2 = [ASnode2_info_type, ASnode2_AS_tier, ASnode2_info_traffic, ASnode2_info_ratio, ASnode2_info_scope, ASnode2_policy_general, ASnode2_policy_locations, ASnode2_policy_ratio, ASnode2_policy_contracts]
    c1 = jnp.pad(jnp.stack([x.astype(jnp.int32) for x in cats1], axis=1),
                 ((0, 0), (0, 16 - 9)))
    c2 = jnp.pad(jnp.stack([x.astype(jnp.int32) for x in cats2], axis=1),
                 ((0, 0), (0, 16 - 9)))

    tabs1 = [t1_info_type, t1_AS_tier, t1_info_traffic, t1_info_ratio, t1_info_scope, t1_policy_general, t1_policy_locations, t1_policy_ratio, t1_policy_contracts]
    tabs2 = [t2_info_type, t2_AS_tier, t2_info_traffic, t2_info_ratio, t2_info_scope, t2_policy_general, t2_policy_locations, t2_policy_ratio, t2_policy_contracts]
    t1b = jax.scipy.linalg.block_diag(*tabs1)
    t2b = jax.scipy.linalg.block_diag(*tabs2)

    w = W[:, 0]
    wmain = w[0:64].reshape(1, 64)
    wcat = w[64:128].reshape(1, 64)
    wixp = jnp.pad(w[128:143], (0, 1)).reshape(_DIXP + 1, 1)
    wfac = jnp.pad(w[143:163], (0, 4)).reshape(24, 1)

    logits = _tc(u_main, v_main, s1x, s2x, s1f, s2f, c1, c2, t1b, t2b,
                 wmain, wcat, wixp, wfac, b.reshape(1, 1))
    return logits.reshape(_B, 1)


# speedup vs baseline: 1.1613x; 1.1613x over previous
import functools

import jax
import jax.numpy as jnp
from jax import lax
from jax.experimental import pallas as pl
from jax.experimental.pallas import tpu as pltpu
from jax.experimental.pallas import tpu_sc as plsc

_B = 16384
_SUB = 256
_NCHUNK = _B // _SUB
_GRP = _SUB // 16
_L = 50
_DIXP = 15
_DFAC = 20
_WIXP = 9
_WFAC = 11
_BLK = 2048

_CAT_OFFS = [0, 8, 16, 32, 40, 48, 52, 56, 60]

_SC_PARAMS = pltpu.CompilerParams(
    needs_layout_passes=False, use_tc_tiling_on_sc=False)

_HI = jnp.uint32(0xFFFF0000)


def _unpack2(word_f32):
    w = plsc.bitcast(word_f32, jnp.uint32)
    even = plsc.bitcast(w << 16, jnp.float32)
    odd = plsc.bitcast(w & _HI, jnp.float32)
    return even, odd


def _sc_a_body(emb_u, emb_i, uidx2, iidx2, t1x, t2x, ix1, ix2,
               u_out, v_out, s1x_out, s2x_out,
               tix_v, idx_v, rows_v, it_v, soutx_v, sem):
    c = lax.axis_index("c")
    s = lax.axis_index("s")
    wid = s * 2 + c

    for (emb, midx2, tixp, ixraw, mout, sout) in (
            (emb_u, uidx2, t1x, ix1, u_out, s1x_out),
            (emb_i, iidx2, t2x, ix2, v_out, s2x_out)):
        pltpu.sync_copy(tixp, tix_v)
        pltpu.sync_copy(midx2.at[pl.ds(wid * 4, 4)], idx_v)
        cps = [pltpu.async_copy(emb.at[idx_v.at[mc]],
                                rows_v.at[pl.ds(mc * 128, 128)], sem)
               for mc in range(4)]

        for sub in range(2):
            chunk = wid * 2 + sub
            pltpu.sync_copy(ixraw.at[chunk], it_v)

            def group(g, carry):
                o = g * 16

                def kstep(k2, accs):
                    accs = list(accs)
                    for dk in range(2):
                        k = k2 * 2 + dk
                        rbase = it_v[k, pl.ds(o, 16)] * _WIXP
                        for jp in range(8):
                            ev, od = _unpack2(
                                plsc.load_gather(tix_v, [rbase + jp]))
                            accs[2 * jp] = accs[2 * jp] + ev
                            if 2 * jp + 1 < _DIXP:
                                accs[2 * jp + 1] = accs[2 * jp + 1] + od
                    return tuple(accs)

                accs = lax.fori_loop(
                    0, _L // 2, kstep,
                    tuple(jnp.zeros((16,), jnp.float32)
                          for _ in range(_DIXP)))
                for j in range(_DIXP):
                    soutx_v[j, pl.ds(o, 16)] = accs[j]
                soutx_v[_DIXP, pl.ds(o, 16)] = jnp.zeros((16,), jnp.float32)
                return carry

            lax.fori_loop(0, _GRP, group, 0)
            pltpu.sync_copy(soutx_v, sout.at[chunk])

        for cp in cps:
            cp.wait()
        pltpu.sync_copy(rows_v, mout.at[pl.ds(wid * 512, 512)])


def _sc_a(emb_u, emb_i, uidx2, iidx2, t1x, t2x, ix1, ix2):
    mesh = plsc.VectorSubcoreMesh(core_axis_name="c", subcore_axis_name="s")
    f = functools.partial(
        pl.kernel, mesh=mesh, compiler_params=_SC_PARAMS,
        out_type=(
            jax.ShapeDtypeStruct((_B, 64), jnp.float32),
            jax.ShapeDtypeStruct((_B, 64), jnp.float32),
            jax.ShapeDtypeStruct((_NCHUNK, _DIXP + 1, _SUB), jnp.float32),
            jax.ShapeDtypeStruct((_NCHUNK, _DIXP + 1, _SUB), jnp.float32),
        ),
        scratch_types=(
            pltpu.VMEM((1000 * _WIXP,), jnp.float32),
            pltpu.VMEM((4, 128), jnp.int32),
            pltpu.VMEM((512, 64), jnp.float32),
            pltpu.VMEM((_L, _SUB), jnp.int32),
            pltpu.VMEM((_DIXP + 1, _SUB), jnp.float32),
            pltpu.SemaphoreType.DMA,
        ),
    )(_sc_a_body)
    return f(emb_u, emb_i, uidx2, iidx2, t1x, t2x, ix1, ix2)


def _sc_b_body(t1f, t2f, if1, if2, s1f_out, s2f_out,
               tfac_v, it_v, soutf_v):
    c = lax.axis_index("c")
    s = lax.axis_index("s")

    def fac_side(tfac, ifraw, sout):
        pltpu.sync_copy(tfac, tfac_v)

        def do_sub(sub, carry0):
            chunk = s * 4 + sub
            pltpu.sync_copy(ifraw.at[chunk], it_v)

            def group(g, carry):
                o = g * 16

                def kstep(k2, accs):
                    accs = list(accs)
                    for dk in range(2):
                        k = k2 * 2 + dk
                        rbase = it_v[k, pl.ds(o, 16)] * _WFAC
                        for jp in range(10):
                            ev, od = _unpack2(
                                plsc.load_gather(tfac_v, [rbase + jp]))
                            accs[2 * jp] = accs[2 * jp] + ev
                            accs[2 * jp + 1] = accs[2 * jp + 1] + od
                    return tuple(accs)

                accs = lax.fori_loop(
                    0, _L // 2, kstep,
                    tuple(jnp.zeros((16,), jnp.float32)
                          for _ in range(_DFAC)))
                for j in range(_DFAC):
                    soutf_v[j, pl.ds(o, 16)] = accs[j]
                for j in range(_DFAC, 24):
                    soutf_v[j, pl.ds(o, 16)] = jnp.zeros((16,), jnp.float32)
                return carry

            lax.fori_loop(0, _GRP, group, 0)
            pltpu.sync_copy(soutf_v, sout.at[chunk])
            return carry0

        lax.fori_loop(0, 4, do_sub, 0)

    @pl.when(c == 0)
    def _():
        fac_side(t1f, if1, s1f_out)

    @pl.when(c == 1)
    def _():
        fac_side(t2f, if2, s2f_out)


def _sc_b(t1f, t2f, if1, if2):
    mesh = plsc.VectorSubcoreMesh(core_axis_name="c", subcore_axis_name="s")
    f = functools.partial(
        pl.kernel, mesh=mesh, compiler_params=_SC_PARAMS,
        out_type=(
            jax.ShapeDtypeStruct((_NCHUNK, 24, _SUB), jnp.float32),
            jax.ShapeDtypeStruct((_NCHUNK, 24, _SUB), jnp.float32),
        ),
        scratch_types=(
            pltpu.VMEM((5000 * _WFAC,), jnp.float32),
            pltpu.VMEM((_L, _SUB), jnp.int32),
            pltpu.VMEM((24, _SUB), jnp.float32),
        ),
    )(_sc_b_body)
    return f(t1f, t2f, if1, if2)


def _tc_body(u_ref, v_ref, s1x_ref, s2x_ref, s1f_ref, s2f_ref,
             c1_ref, c2_ref, t1b_ref, t2b_ref,
             wmain_ref, wcat_ref, wixp_ref, wfac_ref, b_ref, out_ref):
    u = u_ref[...]
    v = v_ref[...]
    mterm = jnp.sum(u * v * wmain_ref[0, :][None, :], axis=1)

    lane = lax.broadcasted_iota(jnp.int32, (_BLK, 64), 1)
    oh1 = jnp.zeros((_BLK, 64), jnp.float32)
    oh2 = jnp.zeros((_BLK, 64), jnp.float32)
    for ci in range(9):
        off = _CAT_OFFS[ci]
        oh1 = oh1 + (lane == (c1_ref[:, ci][:, None] + off)).astype(jnp.float32)
        oh2 = oh2 + (lane == (c2_ref[:, ci][:, None] + off)).astype(jnp.float32)
    ucat = jnp.dot(oh1, t1b_ref[...], preferred_element_type=jnp.float32)
    vcat = jnp.dot(oh2, t2b_ref[...], preferred_element_type=jnp.float32)
    cterm = jnp.sum(ucat * vcat * wcat_ref[0, :][None, :], axis=1)

    xterm = jnp.sum(s1x_ref[...] * s2x_ref[...] * wixp_ref[...],
                    axis=0) * (1.0 / 2500.0)
    fterm = jnp.sum(s1f_ref[...] * s2f_ref[...] * wfac_ref[...],
                    axis=0) * (1.0 / 2500.0)

    out_ref[...] = mterm + cterm + xterm + fterm + b_ref[0, 0]


def _tc(u_main, v_main, s1x, s2x, s1f, s2f, c1, c2, t1b, t2b,
        wmain, wcat, wixp, wfac, b):
    grid = (_B // _BLK,)
    return pl.pallas_call(
        _tc_body,
        grid=grid,
        in_specs=[
            pl.BlockSpec((_BLK, 64), lambda i: (i, 0)),
            pl.BlockSpec((_BLK, 64), lambda i: (i, 0)),
            pl.BlockSpec((_DIXP + 1, _BLK), lambda i: (0, i)),
            pl.BlockSpec((_DIXP + 1, _BLK), lambda i: (0, i)),
            pl.BlockSpec((24, _BLK), lambda i: (0, i)),
            pl.BlockSpec((24, _BLK), lambda i: (0, i)),
            pl.BlockSpec((_BLK, 16), lambda i: (i, 0)),
            pl.BlockSpec((_BLK, 16), lambda i: (i, 0)),
            pl.BlockSpec((64, 64), lambda i: (0, 0)),
            pl.BlockSpec((64, 64), lambda i: (0, 0)),
            pl.BlockSpec((1, 64), lambda i: (0, 0)),
            pl.BlockSpec((1, 64), lambda i: (0, 0)),
            pl.BlockSpec((_DIXP + 1, 1), lambda i: (0, 0)),
            pl.BlockSpec((24, 1), lambda i: (0, 0)),
            pl.BlockSpec((1, 1), lambda i: (0, 0)),
        ],
        out_specs=pl.BlockSpec((_BLK,), lambda i: (i,)),
        out_shape=jax.ShapeDtypeStruct((_B,), jnp.float32),
    )(u_main, v_main, s1x, s2x, s1f, s2f, c1, c2, t1b, t2b,
      wmain, wcat, wixp, wfac, b)


def _untranspose(s3, rows):
    return jnp.transpose(s3, (1, 0, 2)).reshape(rows, _B)


def _chunked(idx2d):
    t = jnp.transpose(idx2d.astype(jnp.int32), (1, 0))
    t = t.reshape(_L, _NCHUNK, _SUB)
    return jnp.transpose(t, (1, 0, 2))


def _pack_bf16(tab, words):
    n, d = tab.shape
    if d % 2:
        tab = jnp.pad(tab, ((0, 0), (0, 1)))
        d += 1
    ev = jax.lax.bitcast_convert_type(
        tab[:, 0::2].astype(jnp.bfloat16), jnp.uint16).astype(jnp.uint32)
    od = jax.lax.bitcast_convert_type(
        tab[:, 1::2].astype(jnp.bfloat16), jnp.uint16).astype(jnp.uint32)
    w = jax.lax.bitcast_convert_type(ev | (od << 16), jnp.float32)
    w = jnp.pad(w, ((0, 0), (0, words - d // 2)))
    return w.reshape(-1)


def kernel(user_indices, item_indices, ASnode1_info_type, ASnode1_AS_tier, ASnode1_info_traffic, ASnode1_info_ratio, ASnode1_info_scope, ASnode1_policy_general, ASnode1_policy_locations, ASnode1_policy_ratio, ASnode1_policy_contracts, ASnode1_appearIXP, ASnode1_appearFac, ASnode2_info_type, ASnode2_AS_tier, ASnode2_info_traffic, ASnode2_info_ratio, ASnode2_info_scope, ASnode2_policy_general, ASnode2_policy_locations, ASnode2_policy_ratio, ASnode2_policy_contracts, ASnode2_appearIXP, ASnode2_appearFac, emb_user, emb_item, t1_info_type, t1_AS_tier, t1_info_traffic, t1_info_ratio, t1_info_scope, t1_policy_general, t1_policy_locations, t1_policy_ratio, t1_policy_contracts, t1_appearIXP, t1_appearFac, t2_info_type, t2_AS_tier, t2_info_traffic, t2_info_ratio, t2_info_scope, t2_policy_general, t2_policy_locations, t2_policy_ratio, t2_policy_contracts, t2_appearIXP, t2_appearFac, W, b):
    uidx2 = user_indices.astype(jnp.int32).reshape(_B // 128, 128)
    iidx2 = item_indices.astype(jnp.int32).reshape(_B // 128, 128)
    ix1 = _chunked(ASnode1_appearIXP)
    ix2 = _chunked(ASnode2_appearIXP)
    if1 = _chunked(ASnode1_appearFac)
    if2 = _chunked(ASnode2_appearFac)
    t1x = _pack_bf16(t1_appearIXP, _WIXP)
    t2x = _pack_bf16(t2_appearIXP, _WIXP)
    t1f = _pack_bf16(t1_appearFac, _WFAC)
    t2f = _pack_bf16(t2_appearFac, _WFAC)

    u_main, v_main, s1x3, s2x3 = _sc_a(
        emb_user, emb_item, uidx2, iidx2, t1x, t2x, ix1, ix2)
    s1f3, s2f3 = _sc_b(t1f, t2f, if1, if2)
    s1x = _untranspose(s1x3, _DIXP + 1)
    s2x = _untranspose(s2x3, _DIXP + 1)
    s1f = _untranspose(s1f3, 24)
    s2f = _untranspose(s2f3, 24)

    cats1 = [ASnode1_info_type, ASnode1_AS_tier, ASnode1_info_traffic, ASnode1_info_ratio, ASnode1_info_scope, ASnode1_policy_general, ASnode1_policy_locations, ASnode1_policy_ratio, ASnode1_policy_contracts]
    cats2 = [ASnode2_info_type, ASnode2_AS_tier, ASnode2_info_traffic, ASnode2_info_ratio, ASnode2_info_scope, ASnode2_policy_general, ASnode2_policy_locations, ASnode2_policy_ratio, ASnode2_policy_contracts]
    c1 = jnp.pad(jnp.stack([x.astype(jnp.int32) for x in cats1], axis=1),
                 ((0, 0), (0, 16 - 9)))
    c2 = jnp.pad(jnp.stack([x.astype(jnp.int32) for x in cats2], axis=1),
                 ((0, 0), (0, 16 - 9)))

    tabs1 = [t1_info_type, t1_AS_tier, t1_info_traffic, t1_info_ratio, t1_info_scope, t1_policy_general, t1_policy_locations, t1_policy_ratio, t1_policy_contracts]
    tabs2 = [t2_info_type, t2_AS_tier, t2_info_traffic, t2_info_ratio, t2_info_scope, t2_policy_general, t2_policy_locations, t2_policy_ratio, t2_policy_contracts]
    t1b = jax.scipy.linalg.block_diag(*tabs1)
    t2b = jax.scipy.linalg.block_diag(*tabs2)

    w = W[:, 0]
    wmain = w[0:64].reshape(1, 64)
    wcat = w[64:128].reshape(1, 64)
    wixp = jnp.pad(w[128:143], (0, 1)).reshape(_DIXP + 1, 1)
    wfac = jnp.pad(w[143:163], (0, 4)).reshape(24, 1)

    logits = _tc(u_main, v_main, s1x, s2x, s1f, s2f, c1, c2, t1b, t2b,
                 wmain, wcat, wixp, wfac, b.reshape(1, 1))
    return logits.reshape(_B, 1)


# speedup vs baseline: 1.1627x; 1.0012x over previous
import functools

import jax
import jax.numpy as jnp
from jax import lax
from jax.experimental import pallas as pl
from jax.experimental.pallas import tpu as pltpu
from jax.experimental.pallas import tpu_sc as plsc

_B = 16384
_SUB = 256
_NCHUNK = _B // _SUB
_GRP = _SUB // 16
_L = 50
_DIXP = 15
_DFAC = 20
_WIXP = 9
_WFAC = 11
_BLK = 2048

_CAT_OFFS = [0, 8, 16, 32, 40, 48, 52, 56, 60]

_SC_PARAMS = pltpu.CompilerParams(
    needs_layout_passes=False, use_tc_tiling_on_sc=False)

_HI = jnp.uint32(0xFFFF0000)


def _unpack2(word_f32):
    w = plsc.bitcast(word_f32, jnp.uint32)
    even = plsc.bitcast(w << 16, jnp.float32)
    odd = plsc.bitcast(w & _HI, jnp.float32)
    return even, odd


def _sc_a_body(emb_u, emb_i, uidx2, iidx2, t1x, t2x, ix1, ix2,
               u_out, v_out, s1x_out, s2x_out,
               tix_v, idx_v, rows_v, it_v, soutx_v, sem):
    c = lax.axis_index("c")
    s = lax.axis_index("s")
    wid = s * 2 + c

    for (emb, midx2, tixp, ixraw, mout, sout) in (
            (emb_u, uidx2, t1x, ix1, u_out, s1x_out),
            (emb_i, iidx2, t2x, ix2, v_out, s2x_out)):
        pltpu.sync_copy(tixp, tix_v)
        pltpu.sync_copy(midx2.at[pl.ds(wid * 4, 4)], idx_v)
        cps = [pltpu.async_copy(emb.at[idx_v.at[mc]],
                                rows_v.at[pl.ds(mc * 128, 128)], sem)
               for mc in range(4)]

        for sub in range(2):
            chunk = wid * 2 + sub
            pltpu.sync_copy(ixraw.at[chunk], it_v)

            def group(g, carry):
                o = g * 16

                def kstep(k2, accs):
                    accs = list(accs)
                    for dk in range(2):
                        k = k2 * 2 + dk
                        rbase = it_v[k, pl.ds(o, 16)] * _WIXP
                        for jp in range(8):
                            ev, od = _unpack2(
                                plsc.load_gather(tix_v, [rbase + jp]))
                            accs[2 * jp] = accs[2 * jp] + ev
                            if 2 * jp + 1 < _DIXP:
                                accs[2 * jp + 1] = accs[2 * jp + 1] + od
                    return tuple(accs)

                accs = lax.fori_loop(
                    0, _L // 2, kstep,
                    tuple(jnp.zeros((16,), jnp.float32)
                          for _ in range(_DIXP)))
                for j in range(_DIXP):
                    soutx_v[j, pl.ds(o, 16)] = accs[j]
                soutx_v[_DIXP, pl.ds(o, 16)] = jnp.zeros((16,), jnp.float32)
                return carry

            lax.fori_loop(0, _GRP, group, 0)
            pltpu.sync_copy(soutx_v, sout.at[chunk])

        for cp in cps:
            cp.wait()
        pltpu.sync_copy(rows_v, mout.at[pl.ds(wid * 512, 512)])


def _sc_a(emb_u, emb_i, uidx2, iidx2, t1x, t2x, ix1, ix2):
    mesh = plsc.VectorSubcoreMesh(core_axis_name="c", subcore_axis_name="s")
    f = functools.partial(
        pl.kernel, mesh=mesh, compiler_params=_SC_PARAMS,
        out_type=(
            jax.ShapeDtypeStruct((_B, 64), jnp.float32),
            jax.ShapeDtypeStruct((_B, 64), jnp.float32),
            jax.ShapeDtypeStruct((_NCHUNK, _DIXP + 1, _SUB), jnp.float32),
            jax.ShapeDtypeStruct((_NCHUNK, _DIXP + 1, _SUB), jnp.float32),
        ),
        scratch_types=(
            pltpu.VMEM((1000 * _WIXP,), jnp.float32),
            pltpu.VMEM((4, 128), jnp.int32),
            pltpu.VMEM((512, 64), jnp.float32),
            pltpu.VMEM((_L, _SUB), jnp.int32),
            pltpu.VMEM((_DIXP + 1, _SUB), jnp.float32),
            pltpu.SemaphoreType.DMA,
        ),
    )(_sc_a_body)
    return f(emb_u, emb_i, uidx2, iidx2, t1x, t2x, ix1, ix2)


def _sc_b_body(t1f, t2f, if1, if2, s1f_out, s2f_out,
               tfac_v, it_v, soutf_v):
    c = lax.axis_index("c")
    s = lax.axis_index("s")

    def fac_side(tfac, ifraw, sout):
        pltpu.sync_copy(tfac, tfac_v)

        def do_sub(sub, carry0):
            chunk = s * 4 + sub
            pltpu.sync_copy(ifraw.at[chunk], it_v)

            def group(g, carry):
                o = g * 16

                def kstep(k2, accs):
                    accs = list(accs)
                    for dk in range(2):
                        k = k2 * 2 + dk
                        rbase = it_v[k, pl.ds(o, 16)] * _WFAC
                        for jp in range(10):
                            ev, od = _unpack2(
                                plsc.load_gather(tfac_v, [rbase + jp]))
                            accs[2 * jp] = accs[2 * jp] + ev
                            accs[2 * jp + 1] = accs[2 * jp + 1] + od
                    return tuple(accs)

                accs = lax.fori_loop(
                    0, _L // 2, kstep,
                    tuple(jnp.zeros((16,), jnp.float32)
                          for _ in range(_DFAC)))
                for j in range(_DFAC):
                    soutf_v[j, pl.ds(o, 16)] = accs[j]
                for j in range(_DFAC, 24):
                    soutf_v[j, pl.ds(o, 16)] = jnp.zeros((16,), jnp.float32)
                return carry

            lax.fori_loop(0, _GRP, group, 0)
            pltpu.sync_copy(soutf_v, sout.at[chunk])
            return carry0

        lax.fori_loop(0, 4, do_sub, 0)

    @pl.when(c == 0)
    def _():
        fac_side(t1f, if1, s1f_out)

    @pl.when(c == 1)
    def _():
        fac_side(t2f, if2, s2f_out)


def _sc_b(t1f, t2f, if1, if2):
    mesh = plsc.VectorSubcoreMesh(core_axis_name="c", subcore_axis_name="s")
    f = functools.partial(
        pl.kernel, mesh=mesh, compiler_params=_SC_PARAMS,
        out_type=(
            jax.ShapeDtypeStruct((_NCHUNK, 24, _SUB), jnp.float32),
            jax.ShapeDtypeStruct((_NCHUNK, 24, _SUB), jnp.float32),
        ),
        scratch_types=(
            pltpu.VMEM((5000 * _WFAC,), jnp.float32),
            pltpu.VMEM((_L, _SUB), jnp.int32),
            pltpu.VMEM((24, _SUB), jnp.float32),
        ),
    )(_sc_b_body)
    return f(t1f, t2f, if1, if2)


def _tc_body(u_ref, v_ref, s1x_ref, s2x_ref, s1f_ref, s2f_ref,
             c1_ref, c2_ref, t1b_ref, t2b_ref,
             wmain_ref, wcat_ref, wixp_ref, wfac_ref, b_ref, out_ref):
    u = u_ref[...]
    v = v_ref[...]
    mterm = jnp.sum(u * v * wmain_ref[0, :][None, :], axis=1)

    lane = lax.broadcasted_iota(jnp.int32, (_BLK, 64), 1)
    oh1 = jnp.zeros((_BLK, 64), jnp.float32)
    oh2 = jnp.zeros((_BLK, 64), jnp.float32)
    for ci in range(9):
        off = _CAT_OFFS[ci]
        oh1 = oh1 + (lane == (c1_ref[:, ci][:, None] + off)).astype(jnp.float32)
        oh2 = oh2 + (lane == (c2_ref[:, ci][:, None] + off)).astype(jnp.float32)
    ucat = jnp.dot(oh1, t1b_ref[...], preferred_element_type=jnp.float32)
    vcat = jnp.dot(oh2, t2b_ref[...], preferred_element_type=jnp.float32)
    cterm = jnp.sum(ucat * vcat * wcat_ref[0, :][None, :], axis=1)

    xterm = jnp.sum(s1x_ref[...] * s2x_ref[...] * wixp_ref[...],
                    axis=0) * (1.0 / 2500.0)
    fterm = jnp.sum(s1f_ref[...] * s2f_ref[...] * wfac_ref[...],
                    axis=0) * (1.0 / 2500.0)

    out_ref[...] = mterm + cterm + xterm + fterm + b_ref[0, 0]


def _tc(u_main, v_main, s1x, s2x, s1f, s2f, c1, c2, t1b, t2b,
        wmain, wcat, wixp, wfac, b):
    grid = (_B // _BLK,)
    return pl.pallas_call(
        _tc_body,
        grid=grid,
        in_specs=[
            pl.BlockSpec((_BLK, 64), lambda i: (i, 0)),
            pl.BlockSpec((_BLK, 64), lambda i: (i, 0)),
            pl.BlockSpec((_DIXP + 1, _BLK), lambda i: (0, i)),
            pl.BlockSpec((_DIXP + 1, _BLK), lambda i: (0, i)),
            pl.BlockSpec((24, _BLK), lambda i: (0, i)),
            pl.BlockSpec((24, _BLK), lambda i: (0, i)),
            pl.BlockSpec((_BLK, 16), lambda i: (i, 0)),
            pl.BlockSpec((_BLK, 16), lambda i: (i, 0)),
            pl.BlockSpec((64, 64), lambda i: (0, 0)),
            pl.BlockSpec((64, 64), lambda i: (0, 0)),
            pl.BlockSpec((1, 64), lambda i: (0, 0)),
            pl.BlockSpec((1, 64), lambda i: (0, 0)),
            pl.BlockSpec((_DIXP + 1, 1), lambda i: (0, 0)),
            pl.BlockSpec((24, 1), lambda i: (0, 0)),
            pl.BlockSpec((1, 1), lambda i: (0, 0)),
        ],
        out_specs=pl.BlockSpec((_BLK,), lambda i: (i,)),
        out_shape=jax.ShapeDtypeStruct((_B,), jnp.float32),
    )(u_main, v_main, s1x, s2x, s1f, s2f, c1, c2, t1b, t2b,
      wmain, wcat, wixp, wfac, b)


def _untranspose(s3, rows):
    return jnp.transpose(s3, (1, 0, 2)).reshape(rows, _B)


def _chunked(idx2d):
    t = idx2d.astype(jnp.int32).reshape(_NCHUNK, _SUB, _L)
    return jnp.transpose(t, (0, 2, 1))


def _pack_bf16(tab, words):
    n, d = tab.shape
    if d % 2:
        tab = jnp.pad(tab, ((0, 0), (0, 1)))
        d += 1
    ev = jax.lax.bitcast_convert_type(
        tab[:, 0::2].astype(jnp.bfloat16), jnp.uint16).astype(jnp.uint32)
    od = jax.lax.bitcast_convert_type(
        tab[:, 1::2].astype(jnp.bfloat16), jnp.uint16).astype(jnp.uint32)
    w = jax.lax.bitcast_convert_type(ev | (od << 16), jnp.float32)
    w = jnp.pad(w, ((0, 0), (0, words - d // 2)))
    return w.reshape(-1)


def kernel(user_indices, item_indices, ASnode1_info_type, ASnode1_AS_tier, ASnode1_info_traffic, ASnode1_info_ratio, ASnode1_info_scope, ASnode1_policy_general, ASnode1_policy_locations, ASnode1_policy_ratio, ASnode1_policy_contracts, ASnode1_appearIXP, ASnode1_appearFac, ASnode2_info_type, ASnode2_AS_tier, ASnode2_info_traffic, ASnode2_info_ratio, ASnode2_info_scope, ASnode2_policy_general, ASnode2_policy_locations, ASnode2_policy_ratio, ASnode2_policy_contracts, ASnode2_appearIXP, ASnode2_appearFac, emb_user, emb_item, t1_info_type, t1_AS_tier, t1_info_traffic, t1_info_ratio, t1_info_scope, t1_policy_general, t1_policy_locations, t1_policy_ratio, t1_policy_contracts, t1_appearIXP, t1_appearFac, t2_info_type, t2_AS_tier, t2_info_traffic, t2_info_ratio, t2_info_scope, t2_policy_general, t2_policy_locations, t2_policy_ratio, t2_policy_contracts, t2_appearIXP, t2_appearFac, W, b):
    uidx2 = user_indices.astype(jnp.int32).reshape(_B // 128, 128)
    iidx2 = item_indices.astype(jnp.int32).reshape(_B // 128, 128)
    ix1 = _chunked(ASnode1_appearIXP)
    ix2 = _chunked(ASnode2_appearIXP)
    if1 = _chunked(ASnode1_appearFac)
    if2 = _chunked(ASnode2_appearFac)
    t1x = _pack_bf16(t1_appearIXP, _WIXP)
    t2x = _pack_bf16(t2_appearIXP, _WIXP)
    t1f = _pack_bf16(t1_appearFac, _WFAC)
    t2f = _pack_bf16(t2_appearFac, _WFAC)

    u_main, v_main, s1x3, s2x3 = _sc_a(
        emb_user, emb_item, uidx2, iidx2, t1x, t2x, ix1, ix2)
    s1f3, s2f3 = _sc_b(t1f, t2f, if1, if2)
    s1x = _untranspose(s1x3, _DIXP + 1)
    s2x = _untranspose(s2x3, _DIXP + 1)
    s1f = _untranspose(s1f3, 24)
    s2f = _untranspose(s2f3, 24)

    cats1 = [ASnode1_info_type, ASnode1_AS_tier, ASnode1_info_traffic, ASnode1_info_ratio, ASnode1_info_scope, ASnode1_policy_general, ASnode1_policy_locations, ASnode1_policy_ratio, ASnode1_policy_contracts]
    cats2 = [ASnode2_info_type, ASnode2_AS_tier, ASnode2_info_traffic, ASnode2_info_ratio, ASnode2_info_scope, ASnode2_policy_general, ASnode2_policy_locations, ASnode2_policy_ratio, ASnode2_policy_contracts]
    c1 = jnp.pad(jnp.stack([x.astype(jnp.int32) for x in cats1], axis=1),
                 ((0, 0), (0, 16 - 9)))
    c2 = jnp.pad(jnp.stack([x.astype(jnp.int32) for x in cats2], axis=1),
                 ((0, 0), (0, 16 - 9)))

    tabs1 = [t1_info_type, t1_AS_tier, t1_info_traffic, t1_info_ratio, t1_info_scope, t1_policy_general, t1_policy_locations, t1_policy_ratio, t1_policy_contracts]
    tabs2 = [t2_info_type, t2_AS_tier, t2_info_traffic, t2_info_ratio, t2_info_scope, t2_policy_general, t2_policy_locations, t2_policy_ratio, t2_policy_contracts]
    t1b = jax.scipy.linalg.block_diag(*tabs1)
    t2b = jax.scipy.linalg.block_diag(*tabs2)

    w = W[:, 0]
    wmain = w[0:64].reshape(1, 64)
    wcat = w[64:128].reshape(1, 64)
    wixp = jnp.pad(w[128:143], (0, 1)).reshape(_DIXP + 1, 1)
    wfac = jnp.pad(w[143:163], (0, 4)).reshape(24, 1)

    logits = _tc(u_main, v_main, s1x, s2x, s1f, s2f, c1, c2, t1b, t2b,
                 wmain, wcat, wixp, wfac, b.reshape(1, 1))
    return logits.reshape(_B, 1)
